# baseline (speedup 1.0000x reference)
import functools

import jax
import jax.numpy as jnp
from jax import lax
from jax.experimental import pallas as pl
from jax.experimental.pallas import tpu as pltpu
from jax.experimental.pallas import tpu_sc as plsc

V = 1000
VP = 1024
D = 1024
B = 4
S = 8192

NC = 2
NS = 16
NW = NC * NS
VB = 8
NVS = V // VB
NJOBS = B * NVS
KMAX = (NJOBS + NW - 1) // NW
SC = 1024
NSC = S // SC
LANES = 16


def _proj_body(w_ref, e_ref, b_ref, o_ref):
    o_ref[:, :V] = lax.dot_general(
        w_ref[...], e_ref[...],
        dimension_numbers=(((1,), (1,)), ((), ())),
        preferred_element_type=jnp.float32,
        precision=lax.Precision.HIGHEST,
    ) + b_ref[...]


def _proj(W, embed, bcol):
    return pl.pallas_call(
        _proj_body,
        out_shape=jax.ShapeDtypeStruct((V, VP), jnp.float32),
    )(W, embed, bcol)


@functools.partial(
    pl.kernel,
    mesh=plsc.VectorSubcoreMesh(core_axis_name="c", subcore_axis_name="s",
                                num_cores=NC),
    compiler_params=pltpu.CompilerParams(needs_layout_passes=False),
    out_type=jax.ShapeDtypeStruct((B * V, S), jnp.float32),
    scratch_types=[
        pltpu.VMEM((VB, VP), jnp.float32),
        pltpu.VMEM((VB, VP), jnp.float32),
        pltpu.VMEM((S,), jnp.int32),
        pltpu.VMEM((S,), jnp.int32),
        pltpu.VMEM((VB, SC), jnp.float32),
        pltpu.VMEM((VB, SC), jnp.float32),
        pltpu.SemaphoreType.DMA,
        pltpu.SemaphoreType.DMA,
        pltpu.SemaphoreType.DMA,
        pltpu.SemaphoreType.DMA,
    ],
)
def _lookup(x_hbm, pt_hbm, out_hbm, pt0, pt1, xb0, xb1, st0, st1,
            i0, i1, o0, o1):
    wid = lax.axis_index("s") * NC + lax.axis_index("c")
    pts = (pt0, pt1)
    xbs = (xb0, xb1)
    stages = (st0, st1)
    isems = (i0, i1)
    osems = (o0, o1)

    def _issue_stage(j, p):
        b = j // NVS
        v0 = (j % NVS) * VB
        pltpu.async_copy(pt_hbm.at[pl.ds(v0, VB)], pts[p], isems[p])
        pltpu.async_copy(x_hbm.at[pl.ds(b * S, S)], xbs[p], isems[p])

    def _wait_stage(p):
        pltpu.make_async_copy(pt_hbm.at[pl.ds(0, VB)], pts[p],
                              isems[p]).wait()
        pltpu.make_async_copy(x_hbm.at[pl.ds(0, S)], xbs[p],
                              isems[p]).wait()

    _issue_stage(wid, 0)

    for k in range(KMAX):
        p = k % 2
        j = wid + k * NW

        def _job(k=k, p=p, j=j):
            _wait_stage(p)
            if k + 1 < KMAX:
                jn = wid + (k + 1) * NW
                if k + 1 == KMAX - 1:
                    @pl.when(jn < NJOBS)
                    def _():
                        _issue_stage(jn, 1 - p)
                else:
                    _issue_stage(jn, 1 - p)

            b = j // NVS
            v0 = (j % NVS) * VB
            r0 = b * V + v0
            xb = xbs[p]

            def chunk_pair(k4, carry):
                for t in (0, 1):
                    c = k4 * 2 + t

                    @pl.when(k4 > 0)
                    def _wait_prev():
                        pltpu.make_async_copy(
                            stages[t],
                            out_hbm.at[pl.ds(r0, VB), pl.ds(0, SC)],
                            osems[t]).wait()

                    def grp(g, carry2, t=t, c=c):
                        x16 = xb[pl.ds(c * SC + g * LANES, LANES)]
                        cols = g * LANES + lax.iota(jnp.int32, LANES)
                        for vr in range(VB):
                            rows = jnp.full((LANES,), vr, jnp.int32)
                            vals = plsc.load_gather(pts[p], [rows, x16])
                            plsc.store_scatter(stages[t], [rows, cols], vals)
                        return carry2

                    lax.fori_loop(0, SC // LANES, grp, 0)
                    pltpu.async_copy(
                        stages[t],
                        out_hbm.at[pl.ds(r0, VB), pl.ds(c * SC, SC)],
                        osems[t])
                return carry

            lax.fori_loop(0, NSC // 2, chunk_pair, 0)
            for t in (0, 1):
                pltpu.make_async_copy(
                    stages[t], out_hbm.at[pl.ds(r0, VB), pl.ds(0, SC)],
                    osems[t]).wait()

        if k == KMAX - 1:
            @pl.when(j < NJOBS)
            def _():
                _job()
        else:
            _job()


def kernel(x, embed, W, b):
    pt = _proj(W, embed, b.reshape(V, 1))
    out_t = _lookup(x.reshape(B * S).astype(jnp.int32), pt)
    return jnp.transpose(out_t.reshape(B, V, S), (0, 2, 1))


# speedup vs baseline: 1.0016x; 1.0016x over previous
import functools

import jax
import jax.numpy as jnp
from jax import lax
from jax.experimental import pallas as pl
from jax.experimental.pallas import tpu as pltpu
from jax.experimental.pallas import tpu_sc as plsc

V = 1000
VP = 1024
D = 1024
B = 4
S = 8192

NC = 2
NS = 16
NW = NC * NS
VB = 8
NVS = V // VB
NJOBS = B * NVS
KMAX = (NJOBS + NW - 1) // NW
SC = 1024
NSC = S // SC
LANES = 16


def _proj_body(w_ref, e_ref, b_ref, o_ref):
    o_ref[:, :V] = lax.dot_general(
        w_ref[...], e_ref[...],
        dimension_numbers=(((1,), (1,)), ((), ())),
        preferred_element_type=jnp.float32,
        precision=lax.Precision.HIGHEST,
    ) + b_ref[...]


def _proj(W, embed, bcol):
    return pl.pallas_call(
        _proj_body,
        out_shape=jax.ShapeDtypeStruct((V, VP), jnp.float32),
    )(W, embed, bcol)


@functools.partial(
    pl.kernel,
    mesh=plsc.VectorSubcoreMesh(core_axis_name="c", subcore_axis_name="s",
                                num_cores=NC),
    compiler_params=pltpu.CompilerParams(needs_layout_passes=False),
    out_type=jax.ShapeDtypeStruct((B * V, S), jnp.float32),
    scratch_types=[
        pltpu.VMEM((VB, VP), jnp.float32),
        pltpu.VMEM((VB, VP), jnp.float32),
        pltpu.VMEM((S,), jnp.int32),
        pltpu.VMEM((S,), jnp.int32),
        pltpu.VMEM((VB, SC), jnp.float32),
        pltpu.VMEM((VB, SC), jnp.float32),
        pltpu.SemaphoreType.DMA,
        pltpu.SemaphoreType.DMA,
        pltpu.SemaphoreType.DMA,
        pltpu.SemaphoreType.DMA,
    ],
)
def _lookup(x_hbm, pt_hbm, out_hbm, pt0, pt1, xb0, xb1, st0, st1,
            i0, i1, o0, o1):
    wid = lax.axis_index("s") * NC + lax.axis_index("c")
    ROWS = [jnp.full((LANES,), vr, jnp.int32) for vr in range(VB)]
    pts = (pt0, pt1)
    xbs = (xb0, xb1)
    stages = (st0, st1)
    isems = (i0, i1)
    osems = (o0, o1)

    def _issue_stage(j, p):
        b = j // NVS
        v0 = (j % NVS) * VB
        pltpu.async_copy(pt_hbm.at[pl.ds(v0, VB)], pts[p], isems[p])
        pltpu.async_copy(x_hbm.at[pl.ds(b * S, S)], xbs[p], isems[p])

    def _wait_stage(p):
        pltpu.make_async_copy(pt_hbm.at[pl.ds(0, VB)], pts[p],
                              isems[p]).wait()
        pltpu.make_async_copy(x_hbm.at[pl.ds(0, S)], xbs[p],
                              isems[p]).wait()

    _issue_stage(wid, 0)

    for k in range(KMAX):
        p = k % 2
        j = wid + k * NW

        def _job(k=k, p=p, j=j):
            _wait_stage(p)
            if k + 1 < KMAX:
                jn = wid + (k + 1) * NW
                if k + 1 == KMAX - 1:
                    @pl.when(jn < NJOBS)
                    def _():
                        _issue_stage(jn, 1 - p)
                else:
                    _issue_stage(jn, 1 - p)

            b = j // NVS
            v0 = (j % NVS) * VB
            r0 = b * V + v0
            xb = xbs[p]

            def chunk_pair(k4, carry):
                for t in (0, 1):
                    c = k4 * 2 + t

                    @pl.when(k4 > 0)
                    def _wait_prev():
                        pltpu.make_async_copy(
                            stages[t],
                            out_hbm.at[pl.ds(r0, VB), pl.ds(0, SC)],
                            osems[t]).wait()

                    def grp(g4, carry2, t=t, c=c):
                        for u in range(4):
                            g = g4 * 4 + u
                            x16 = xb[pl.ds(c * SC + g * LANES, LANES)]
                            for vr in range(VB):
                                vals = plsc.load_gather(
                                    pts[p], [ROWS[vr], x16])
                                stages[t][vr, pl.ds(g * LANES, LANES)] = vals
                        return carry2

                    lax.fori_loop(0, SC // (LANES * 4), grp, 0)
                    pltpu.async_copy(
                        stages[t],
                        out_hbm.at[pl.ds(r0, VB), pl.ds(c * SC, SC)],
                        osems[t])
                return carry

            lax.fori_loop(0, NSC // 2, chunk_pair, 0)
            for t in (0, 1):
                pltpu.make_async_copy(
                    stages[t], out_hbm.at[pl.ds(r0, VB), pl.ds(0, SC)],
                    osems[t]).wait()

        if k == KMAX - 1:
            @pl.when(j < NJOBS)
            def _():
                _job()
        else:
            _job()


def kernel(x, embed, W, b):
    pt = _proj(W, embed, b.reshape(V, 1))
    out_t = _lookup(x.reshape(B * S).astype(jnp.int32), pt)
    return jnp.transpose(out_t.reshape(B, V, S), (0, 2, 1))


# speedup vs baseline: 2.6167x; 2.6126x over previous
import functools

import jax
import jax.numpy as jnp
from jax import lax
from jax.experimental import pallas as pl
from jax.experimental.pallas import tpu as pltpu
from jax.experimental.pallas import tpu_sc as plsc

V = 1000
VP = 1024
D = 1024
B = 4
S = 8192

NC = 2
NS = 16
NW = NC * NS
VB = 8
NVS = V // VB
NJOBS = B * NVS
KMAX = (NJOBS + NW - 1) // NW
SC = 1024
NSC = S // SC
LANES = 16


def _proj_body(w_ref, e_ref, b_ref, o_ref):
    o_ref[:, :V] = lax.dot_general(
        w_ref[...], e_ref[...],
        dimension_numbers=(((1,), (1,)), ((), ())),
        preferred_element_type=jnp.float32,
        precision=lax.Precision.HIGHEST,
    ) + b_ref[...]


def _proj(W, embed, bcol):
    return pl.pallas_call(
        _proj_body,
        out_shape=jax.ShapeDtypeStruct((V, VP), jnp.float32),
    )(W, embed, bcol)


@functools.partial(
    pl.kernel,
    mesh=plsc.VectorSubcoreMesh(core_axis_name="c", subcore_axis_name="s",
                                num_cores=NC),
    compiler_params=pltpu.CompilerParams(needs_layout_passes=False),
    out_type=jax.ShapeDtypeStruct((B * V, S), jnp.float32),
    scratch_types=[
        pltpu.VMEM((VB * VP,), jnp.float32),
        pltpu.VMEM((VB * VP,), jnp.float32),
        pltpu.VMEM((S,), jnp.int32),
        pltpu.VMEM((S,), jnp.int32),
        pltpu.VMEM((VB, SC), jnp.float32),
        pltpu.VMEM((VB, SC), jnp.float32),
        pltpu.SemaphoreType.DMA,
        pltpu.SemaphoreType.DMA,
        pltpu.SemaphoreType.DMA,
        pltpu.SemaphoreType.DMA,
    ],
)
def _lookup(x_hbm, pt_hbm, out_hbm, pt0, pt1, xb0, xb1, st0, st1,
            i0, i1, o0, o1):
    wid = lax.axis_index("s") * NC + lax.axis_index("c")
    VOFF = [jnp.full((LANES,), vr * VP, jnp.int32) for vr in range(VB)]
    pts = (pt0, pt1)
    xbs = (xb0, xb1)
    stages = (st0, st1)
    isems = (i0, i1)
    osems = (o0, o1)

    def _issue_stage(j, p):
        b = j // NVS
        v0 = (j % NVS) * VB
        pltpu.async_copy(pt_hbm.at[pl.ds(v0 * VP, VB * VP)], pts[p],
                        isems[p])
        pltpu.async_copy(x_hbm.at[pl.ds(b * S, S)], xbs[p], isems[p])

    def _wait_stage(p):
        pltpu.make_async_copy(pt_hbm.at[pl.ds(0, VB * VP)], pts[p],
                              isems[p]).wait()
        pltpu.make_async_copy(x_hbm.at[pl.ds(0, S)], xbs[p],
                              isems[p]).wait()

    _issue_stage(wid, 0)

    for k in range(KMAX):
        p = k % 2
        j = wid + k * NW

        def _job(k=k, p=p, j=j):
            _wait_stage(p)
            if k + 1 < KMAX:
                jn = wid + (k + 1) * NW
                if k + 1 == KMAX - 1:
                    @pl.when(jn < NJOBS)
                    def _():
                        _issue_stage(jn, 1 - p)
                else:
                    _issue_stage(jn, 1 - p)

            b = j // NVS
            v0 = (j % NVS) * VB
            r0 = b * V + v0
            xb = xbs[p]

            def chunk_pair(k4, carry):
                for t in (0, 1):
                    c = k4 * 2 + t

                    @pl.when(k4 > 0)
                    def _wait_prev():
                        pltpu.make_async_copy(
                            stages[t],
                            out_hbm.at[pl.ds(r0, VB), pl.ds(0, SC)],
                            osems[t]).wait()

                    @plsc.parallel_loop(0, SC // LANES, unroll=4)
                    def _grp(g, t=t, c=c):
                        x16 = xb[pl.ds(c * SC + g * LANES, LANES)]
                        for vr in range(VB):
                            vals = plsc.load_gather(
                                pts[p], [x16 + VOFF[vr]])
                            stages[t][vr, pl.ds(g * LANES, LANES)] = vals
                    pltpu.async_copy(
                        stages[t],
                        out_hbm.at[pl.ds(r0, VB), pl.ds(c * SC, SC)],
                        osems[t])
                return carry

            lax.fori_loop(0, NSC // 2, chunk_pair, 0)
            for t in (0, 1):
                pltpu.make_async_copy(
                    stages[t], out_hbm.at[pl.ds(r0, VB), pl.ds(0, SC)],
                    osems[t]).wait()

        if k == KMAX - 1:
            @pl.when(j < NJOBS)
            def _():
                _job()
        else:
            _job()


def kernel(x, embed, W, b):
    pt = _proj(W, embed, b.reshape(V, 1)).reshape(V * VP)
    out_t = _lookup(x.reshape(B * S).astype(jnp.int32), pt)
    return jnp.transpose(out_t.reshape(B, V, S), (0, 2, 1))


# speedup vs baseline: 2.7081x; 1.0349x over previous
import functools

import jax
import jax.numpy as jnp
from jax import lax
from jax.experimental import pallas as pl
from jax.experimental.pallas import tpu as pltpu
from jax.experimental.pallas import tpu_sc as plsc

V = 1000
VP = 1024
D = 1024
B = 4
S = 8192

NC = 2
NS = 16
NW = NC * NS
VB = 8
NVS = V // VB
NJOBS = B * NVS
KMAX = (NJOBS + NW - 1) // NW
SC = 2048
NSC = S // SC
LANES = 16


def _proj_body(w_ref, e_ref, b_ref, o_ref):
    o_ref[:, :V] = lax.dot_general(
        w_ref[...], e_ref[...],
        dimension_numbers=(((1,), (1,)), ((), ())),
        preferred_element_type=jnp.float32,
        precision=lax.Precision.HIGHEST,
    ) + b_ref[...]


def _proj(W, embed, bcol):
    return pl.pallas_call(
        _proj_body,
        out_shape=jax.ShapeDtypeStruct((V, VP), jnp.float32),
    )(W, embed, bcol)


@functools.partial(
    pl.kernel,
    mesh=plsc.VectorSubcoreMesh(core_axis_name="c", subcore_axis_name="s",
                                num_cores=NC),
    compiler_params=pltpu.CompilerParams(needs_layout_passes=False),
    out_type=jax.ShapeDtypeStruct((B * V, S), jnp.float32),
    scratch_types=[
        pltpu.VMEM((VB * VP,), jnp.float32),
        pltpu.VMEM((VB * VP,), jnp.float32),
        pltpu.VMEM((S,), jnp.int32),
        pltpu.VMEM((S,), jnp.int32),
        pltpu.VMEM((VB, SC), jnp.float32),
        pltpu.VMEM((VB, SC), jnp.float32),
        pltpu.VMEM((VB, SC), jnp.float32),
        pltpu.VMEM((VB, SC), jnp.float32),
        pltpu.SemaphoreType.DMA,
        pltpu.SemaphoreType.DMA,
        pltpu.SemaphoreType.DMA,
        pltpu.SemaphoreType.DMA,
        pltpu.SemaphoreType.DMA,
        pltpu.SemaphoreType.DMA,
    ],
)
def _lookup(x_hbm, pt_hbm, out_hbm, pt0, pt1, xb0, xb1,
            st0, st1, st2, st3, i0, i1, o0, o1, o2, o3):
    wid = lax.axis_index("s") * NC + lax.axis_index("c")
    VOFF = [jnp.full((LANES,), vr * VP, jnp.int32) for vr in range(VB)]
    pts = (pt0, pt1)
    xbs = (xb0, xb1)
    stages = (st0, st1, st2, st3)
    isems = (i0, i1)
    osems = (o0, o1, o2, o3)

    def _issue_stage(j, p):
        b = j // NVS
        v0 = (j % NVS) * VB
        pltpu.async_copy(pt_hbm.at[pl.ds(v0 * VP, VB * VP)], pts[p],
                        isems[p])
        pltpu.async_copy(x_hbm.at[pl.ds(b * S, S)], xbs[p], isems[p])

    def _wait_stage(p):
        pltpu.make_async_copy(pt_hbm.at[pl.ds(0, VB * VP)], pts[p],
                              isems[p]).wait()
        pltpu.make_async_copy(x_hbm.at[pl.ds(0, S)], xbs[p],
                              isems[p]).wait()

    _issue_stage(wid, 0)

    def _job(k, p, first_round):
        j = wid + k * NW
        _wait_stage(p)
        jn = j + NW
        @pl.when(jn < NJOBS)
        def _():
            _issue_stage(jn, 1 - p)

        b = j // NVS
        v0 = (j % NVS) * VB
        r0 = b * V + v0
        xb = xbs[p]

        for c in range(NSC):
            if not first_round:
                pltpu.make_async_copy(
                    stages[c],
                    out_hbm.at[pl.ds(r0, VB), pl.ds(0, SC)],
                    osems[c]).wait()

            @plsc.parallel_loop(0, SC // LANES, unroll=4)
            def _grp(g, c=c):
                x16 = xb[pl.ds(c * SC + g * LANES, LANES)]
                for vr in range(VB):
                    vals = plsc.load_gather(
                        pts[p], [x16 + VOFF[vr]])
                    stages[c][vr, pl.ds(g * LANES, LANES)] = vals

            pltpu.async_copy(
                stages[c],
                out_hbm.at[pl.ds(r0, VB), pl.ds(c * SC, SC)],
                osems[c])

    _job(0, 0, True)

    def _round_pair(i, carry):
        _job(1 + i * 2, 1, False)
        _job(2 + i * 2, 0, False)
        return carry

    lax.fori_loop(0, (KMAX - 2) // 2, _round_pair, 0)

    @pl.when(wid + (KMAX - 1) * NW < NJOBS)
    def _():
        _job(KMAX - 1, (KMAX - 1) % 2, False)

    for c in range(NSC):
        pltpu.make_async_copy(
            stages[c], out_hbm.at[pl.ds(0, VB), pl.ds(0, SC)],
            osems[c]).wait()


def kernel(x, embed, W, b):
    pt = _proj(W, embed, b.reshape(V, 1)).reshape(V * VP)
    out_t = _lookup(x.reshape(B * S).astype(jnp.int32), pt)
    return jnp.transpose(out_t.reshape(B, V, S), (0, 2, 1))


# speedup vs baseline: 2.7085x; 1.0002x over previous
import functools

import jax
import jax.numpy as jnp
from jax import lax
from jax.experimental import pallas as pl
from jax.experimental.pallas import tpu as pltpu
from jax.experimental.pallas import tpu_sc as plsc

V = 1000
VP = 1024
D = 1024
B = 4
S = 8192

NC = 2
NS = 16
NW = NC * NS
VB = 8
NVS = V // VB
NJOBS = B * NVS
KMAX = (NJOBS + NW - 1) // NW
SC = 2048
NSC = S // SC
LANES = 16


def _proj_body(w_ref, e_ref, b_ref, o_ref):
    o_ref[...] = lax.dot_general(
        w_ref[...], e_ref[...],
        dimension_numbers=(((1,), (1,)), ((), ())),
        preferred_element_type=jnp.float32,
        precision=lax.Precision.HIGHEST,
    ) + b_ref[...]


def _proj(W, embed, bcol):
    return pl.pallas_call(
        _proj_body,
        grid=(2,),
        in_specs=[
            pl.BlockSpec((V, D), lambda j: (0, 0)),
            pl.BlockSpec((VP // 2, D), lambda j: (j, 0)),
            pl.BlockSpec((V, 1), lambda j: (0, 0)),
        ],
        out_specs=pl.BlockSpec((V, VP // 2), lambda j: (0, j)),
        out_shape=jax.ShapeDtypeStruct((V, VP), jnp.float32),
    )(W, embed, bcol)


@functools.partial(
    pl.kernel,
    mesh=plsc.VectorSubcoreMesh(core_axis_name="c", subcore_axis_name="s",
                                num_cores=NC),
    compiler_params=pltpu.CompilerParams(needs_layout_passes=False),
    out_type=jax.ShapeDtypeStruct((B * V, S), jnp.float32),
    scratch_types=[
        pltpu.VMEM((VB * VP,), jnp.float32),
        pltpu.VMEM((VB * VP,), jnp.float32),
        pltpu.VMEM((S,), jnp.int32),
        pltpu.VMEM((S,), jnp.int32),
        pltpu.VMEM((VB, SC), jnp.float32),
        pltpu.VMEM((VB, SC), jnp.float32),
        pltpu.VMEM((VB, SC), jnp.float32),
        pltpu.VMEM((VB, SC), jnp.float32),
        pltpu.SemaphoreType.DMA,
        pltpu.SemaphoreType.DMA,
        pltpu.SemaphoreType.DMA,
        pltpu.SemaphoreType.DMA,
        pltpu.SemaphoreType.DMA,
        pltpu.SemaphoreType.DMA,
    ],
)
def _lookup(x_hbm, pt_hbm, out_hbm, pt0, pt1, xb0, xb1,
            st0, st1, st2, st3, i0, i1, o0, o1, o2, o3):
    wid = lax.axis_index("s") * NC + lax.axis_index("c")
    VOFF = [jnp.full((LANES,), vr * VP, jnp.int32) for vr in range(VB)]
    pts = (pt0, pt1)
    xbs = (xb0, xb1)
    stages = (st0, st1, st2, st3)
    isems = (i0, i1)
    osems = (o0, o1, o2, o3)

    def _issue_stage(j, p):
        b = j // NVS
        v0 = (j % NVS) * VB
        pltpu.async_copy(pt_hbm.at[pl.ds(v0 * VP, VB * VP)], pts[p],
                        isems[p])
        pltpu.async_copy(x_hbm.at[pl.ds(b * S, S)], xbs[p], isems[p])

    def _wait_stage(p):
        pltpu.make_async_copy(pt_hbm.at[pl.ds(0, VB * VP)], pts[p],
                              isems[p]).wait()
        pltpu.make_async_copy(x_hbm.at[pl.ds(0, S)], xbs[p],
                              isems[p]).wait()

    _issue_stage(wid, 0)

    def _job(k, p, first_round):
        j = wid + k * NW
        _wait_stage(p)
        jn = j + NW
        @pl.when(jn < NJOBS)
        def _():
            _issue_stage(jn, 1 - p)

        b = j // NVS
        v0 = (j % NVS) * VB
        r0 = b * V + v0
        xb = xbs[p]

        for c in range(NSC):
            if not first_round:
                pltpu.make_async_copy(
                    stages[c],
                    out_hbm.at[pl.ds(r0, VB), pl.ds(0, SC)],
                    osems[c]).wait()

            @plsc.parallel_loop(0, SC // LANES, unroll=8)
            def _grp(g, c=c):
                x16 = xb[pl.ds(c * SC + g * LANES, LANES)]
                for vr in range(VB):
                    vals = plsc.load_gather(
                        pts[p], [x16 + VOFF[vr]])
                    stages[c][vr, pl.ds(g * LANES, LANES)] = vals

            pltpu.async_copy(
                stages[c],
                out_hbm.at[pl.ds(r0, VB), pl.ds(c * SC, SC)],
                osems[c])

    _job(0, 0, True)

    def _round_pair(i, carry):
        _job(1 + i * 2, 1, False)
        _job(2 + i * 2, 0, False)
        return carry

    lax.fori_loop(0, (KMAX - 2) // 2, _round_pair, 0)

    @pl.when(wid + (KMAX - 1) * NW < NJOBS)
    def _():
        _job(KMAX - 1, (KMAX - 1) % 2, False)

    for c in range(NSC):
        pltpu.make_async_copy(
            stages[c], out_hbm.at[pl.ds(0, VB), pl.ds(0, SC)],
            osems[c]).wait()


def kernel(x, embed, W, b):
    pt = _proj(W, embed, b.reshape(V, 1)).reshape(V * VP)
    out_t = _lookup(x.reshape(B * S).astype(jnp.int32), pt)
    return jnp.transpose(out_t.reshape(B, V, S), (0, 2, 1))
